# baseline (speedup 1.0000x reference)
import jax
import jax.numpy as jnp
from jax.experimental import pallas as pl
from jax.experimental.pallas import tpu as pltpu

IN_F = 128
OUT_F = 128
B = 2
N = 4096
TI = 512


def _fused_kernel(adj_ref, xf_ref, ws_ref, wn_ref, bs_ref, bn_ref, out_ref,
                  xhb_ref):
    i = pl.program_id(0)
    ab = adj_ref[...].astype(jnp.bfloat16)
    deg = jnp.sum(ab.astype(jnp.float32), axis=1, keepdims=True)

    @pl.when(i == 0)
    def _():
        xhb_ref[...] = jnp.concatenate(
            [xf_ref[0:N, :], xf_ref[N:2 * N, :]], axis=1
        ).astype(jnp.bfloat16)

    agg = jnp.dot(ab, xhb_ref[...],
                  preferred_element_type=jnp.float32)
    mean = agg / jnp.maximum(deg, 1.0)
    neigh = jnp.dot(mean, wn_ref[...], preferred_element_type=jnp.float32)
    neigh = neigh + bn_ref[...]
    neigh = jnp.where(deg > 0.0, neigh, 0.0)
    xs = jnp.concatenate(
        [xf_ref[pl.ds(i * TI, TI), :], xf_ref[pl.ds(N + i * TI, TI), :]],
        axis=1)
    self_out = jnp.dot(xs, ws_ref[...], preferred_element_type=jnp.float32)
    self_out = self_out + bs_ref[...]
    res = jnp.maximum(self_out + neigh, 0.0)
    out_ref[0] = res[:, :OUT_F]
    out_ref[1] = res[:, OUT_F:]


@jax.jit
def kernel(x, adj_matrix, W_self, b_self, W_neigh, b_neigh):
    xf = x.reshape(B * N, IN_F)
    zero = jnp.zeros((OUT_F, OUT_F), jnp.float32)
    wbd_self = jnp.block([[W_self.T, zero], [zero, W_self.T]])
    wbd_neigh = jnp.block([[W_neigh.T, zero], [zero, W_neigh.T]])
    bbd_self = jnp.concatenate([b_self, b_self]).reshape(1, B * OUT_F)
    bbd_neigh = jnp.concatenate([b_neigh, b_neigh]).reshape(1, B * OUT_F)

    out = pl.pallas_call(
        _fused_kernel,
        grid=(N // TI,),
        in_specs=[
            pl.BlockSpec((TI, N), lambda i: (i, 0)),
            pl.BlockSpec((B * N, IN_F), lambda i: (0, 0)),
            pl.BlockSpec((B * IN_F, B * OUT_F), lambda i: (0, 0)),
            pl.BlockSpec((B * IN_F, B * OUT_F), lambda i: (0, 0)),
            pl.BlockSpec((1, B * OUT_F), lambda i: (0, 0)),
            pl.BlockSpec((1, B * OUT_F), lambda i: (0, 0)),
        ],
        out_specs=pl.BlockSpec((B, TI, OUT_F), lambda i: (0, i, 0)),
        out_shape=jax.ShapeDtypeStruct((B, N, OUT_F), jnp.float32),
        scratch_shapes=[pltpu.VMEM((N, B * IN_F), jnp.bfloat16)],
    )(adj_matrix, xf, wbd_self, wbd_neigh, bbd_self, bbd_neigh)

    return out


# speedup vs baseline: 1.1161x; 1.1161x over previous
import jax
import jax.numpy as jnp
from jax.experimental import pallas as pl

IN_F = 128
OUT_F = 128
B = 2
N = 4096
TI = 512


def _fused_kernel(adj_ref, xf_ref, ws_ref, wn_ref, bs_ref, bn_ref, out_ref):
    i = pl.program_id(0)
    ab = adj_ref[...].astype(jnp.bfloat16)
    deg = jnp.sum(ab.astype(jnp.float32), axis=1, keepdims=True)
    xhb = jnp.concatenate(
        [xf_ref[0:N, :], xf_ref[N:2 * N, :]], axis=1
    ).astype(jnp.bfloat16)
    agg = jnp.dot(ab, xhb, preferred_element_type=jnp.float32)
    mean = agg / jnp.maximum(deg, 1.0)
    neigh = jnp.dot(mean, wn_ref[...], preferred_element_type=jnp.float32)
    neigh = neigh + bn_ref[...]
    neigh = jnp.where(deg > 0.0, neigh, 0.0)
    xs = jnp.concatenate(
        [xf_ref[pl.ds(i * TI, TI), :], xf_ref[pl.ds(N + i * TI, TI), :]],
        axis=1)
    self_out = jnp.dot(xs, ws_ref[...], preferred_element_type=jnp.float32)
    self_out = self_out + bs_ref[...]
    res = jnp.maximum(self_out + neigh, 0.0)
    out_ref[0] = res[:, :OUT_F]
    out_ref[1] = res[:, OUT_F:]


@jax.jit
def kernel(x, adj_matrix, W_self, b_self, W_neigh, b_neigh):
    xf = x.reshape(B * N, IN_F)
    zero = jnp.zeros((OUT_F, OUT_F), jnp.float32)
    wbd_self = jnp.block([[W_self.T, zero], [zero, W_self.T]])
    wbd_neigh = jnp.block([[W_neigh.T, zero], [zero, W_neigh.T]])
    bbd_self = jnp.concatenate([b_self, b_self]).reshape(1, B * OUT_F)
    bbd_neigh = jnp.concatenate([b_neigh, b_neigh]).reshape(1, B * OUT_F)

    out = pl.pallas_call(
        _fused_kernel,
        grid=(N // TI,),
        in_specs=[
            pl.BlockSpec((TI, N), lambda i: (i, 0)),
            pl.BlockSpec((B * N, IN_F), lambda i: (0, 0)),
            pl.BlockSpec((B * IN_F, B * OUT_F), lambda i: (0, 0)),
            pl.BlockSpec((B * IN_F, B * OUT_F), lambda i: (0, 0)),
            pl.BlockSpec((1, B * OUT_F), lambda i: (0, 0)),
            pl.BlockSpec((1, B * OUT_F), lambda i: (0, 0)),
        ],
        out_specs=pl.BlockSpec((B, TI, OUT_F), lambda i: (0, i, 0)),
        out_shape=jax.ShapeDtypeStruct((B, N, OUT_F), jnp.float32),
    )(adj_matrix, xf, wbd_self, wbd_neigh, bbd_self, bbd_neigh)

    return out


# speedup vs baseline: 1.1216x; 1.0049x over previous
import jax
import jax.numpy as jnp
from jax.experimental import pallas as pl
from jax.experimental.pallas import tpu as pltpu

IN_F = 128
OUT_F = 128
B = 2
N = 4096
TI = 512


def _fused_kernel(adj_ref, xf_ref, ws_ref, wn_ref, bs_ref, bn_ref, out_ref):
    i = pl.program_id(0)
    ab = adj_ref[...].astype(jnp.bfloat16)
    deg = jnp.sum(ab.astype(jnp.float32), axis=1, keepdims=True)
    xhb = jnp.concatenate(
        [xf_ref[0:N, :], xf_ref[N:2 * N, :]], axis=1
    ).astype(jnp.bfloat16)
    agg = jnp.dot(ab, xhb, preferred_element_type=jnp.float32)
    mean = agg / jnp.maximum(deg, 1.0)
    neigh = jnp.dot(mean, wn_ref[...], preferred_element_type=jnp.float32)
    neigh = neigh + bn_ref[...]
    neigh = jnp.where(deg > 0.0, neigh, 0.0)
    xs = jnp.concatenate(
        [xf_ref[pl.ds(i * TI, TI), :], xf_ref[pl.ds(N + i * TI, TI), :]],
        axis=1)
    self_out = jnp.dot(xs, ws_ref[...], preferred_element_type=jnp.float32)
    self_out = self_out + bs_ref[...]
    res = jnp.maximum(self_out + neigh, 0.0)
    out_ref[0] = res[:, :OUT_F]
    out_ref[1] = res[:, OUT_F:]


@jax.jit
def kernel(x, adj_matrix, W_self, b_self, W_neigh, b_neigh):
    xf = x.reshape(B * N, IN_F)
    zero = jnp.zeros((OUT_F, OUT_F), jnp.float32)
    wbd_self = jnp.block([[W_self.T, zero], [zero, W_self.T]])
    wbd_neigh = jnp.block([[W_neigh.T, zero], [zero, W_neigh.T]])
    bbd_self = jnp.concatenate([b_self, b_self]).reshape(1, B * OUT_F)
    bbd_neigh = jnp.concatenate([b_neigh, b_neigh]).reshape(1, B * OUT_F)

    out = pl.pallas_call(
        _fused_kernel,
        grid=(N // TI,),
        in_specs=[
            pl.BlockSpec((TI, N), lambda i: (i, 0)),
            pl.BlockSpec((B * N, IN_F), lambda i: (0, 0)),
            pl.BlockSpec((B * IN_F, B * OUT_F), lambda i: (0, 0)),
            pl.BlockSpec((B * IN_F, B * OUT_F), lambda i: (0, 0)),
            pl.BlockSpec((1, B * OUT_F), lambda i: (0, 0)),
            pl.BlockSpec((1, B * OUT_F), lambda i: (0, 0)),
        ],
        out_specs=pl.BlockSpec((B, TI, OUT_F), lambda i: (0, i, 0)),
        out_shape=jax.ShapeDtypeStruct((B, N, OUT_F), jnp.float32),
        compiler_params=pltpu.CompilerParams(
            dimension_semantics=("parallel",)),
    )(adj_matrix, xf, wbd_self, wbd_neigh, bbd_self, bbd_neigh)

    return out


# speedup vs baseline: 1.1220x; 1.0003x over previous
import jax
import jax.numpy as jnp
from jax.experimental import pallas as pl
from jax.experimental.pallas import tpu as pltpu

IN_F = 128
OUT_F = 128
B = 2
N = 4096
TI = 512


def _fused_kernel(adjl_ref, adjr_ref, xf_ref, ws_ref, wn_ref, bs_ref, bn_ref,
                  out_ref):
    i = pl.program_id(0)
    abl = adjl_ref[...].astype(jnp.bfloat16)
    abr = adjr_ref[...].astype(jnp.bfloat16)
    deg = (jnp.sum(abl.astype(jnp.float32), axis=1, keepdims=True)
           + jnp.sum(abr.astype(jnp.float32), axis=1, keepdims=True))
    xhb = jnp.concatenate(
        [xf_ref[0:N, :], xf_ref[N:2 * N, :]], axis=1
    ).astype(jnp.bfloat16)
    agg = (jnp.dot(abl, xhb[0:N // 2], preferred_element_type=jnp.float32)
           + jnp.dot(abr, xhb[N // 2:N], preferred_element_type=jnp.float32))
    mean = agg / jnp.maximum(deg, 1.0)
    neigh = jnp.dot(mean, wn_ref[...], preferred_element_type=jnp.float32)
    neigh = neigh + bn_ref[...]
    neigh = jnp.where(deg > 0.0, neigh, 0.0)
    xs = jnp.concatenate(
        [xf_ref[pl.ds(i * TI, TI), :], xf_ref[pl.ds(N + i * TI, TI), :]],
        axis=1)
    self_out = jnp.dot(xs, ws_ref[...], preferred_element_type=jnp.float32)
    self_out = self_out + bs_ref[...]
    res = jnp.maximum(self_out + neigh, 0.0)
    out_ref[0] = res[:, :OUT_F]
    out_ref[1] = res[:, OUT_F:]


@jax.jit
def kernel(x, adj_matrix, W_self, b_self, W_neigh, b_neigh):
    xf = x.reshape(B * N, IN_F)
    zero = jnp.zeros((OUT_F, OUT_F), jnp.float32)
    wbd_self = jnp.block([[W_self.T, zero], [zero, W_self.T]])
    wbd_neigh = jnp.block([[W_neigh.T, zero], [zero, W_neigh.T]])
    bbd_self = jnp.concatenate([b_self, b_self]).reshape(1, B * OUT_F)
    bbd_neigh = jnp.concatenate([b_neigh, b_neigh]).reshape(1, B * OUT_F)

    out = pl.pallas_call(
        _fused_kernel,
        grid=(N // TI,),
        in_specs=[
            pl.BlockSpec((TI, N // 2), lambda i: (i, 0)),
            pl.BlockSpec((TI, N // 2), lambda i: (i, 1)),
            pl.BlockSpec((B * N, IN_F), lambda i: (0, 0)),
            pl.BlockSpec((B * IN_F, B * OUT_F), lambda i: (0, 0)),
            pl.BlockSpec((B * IN_F, B * OUT_F), lambda i: (0, 0)),
            pl.BlockSpec((1, B * OUT_F), lambda i: (0, 0)),
            pl.BlockSpec((1, B * OUT_F), lambda i: (0, 0)),
        ],
        out_specs=pl.BlockSpec((B, TI, OUT_F), lambda i: (0, i, 0)),
        out_shape=jax.ShapeDtypeStruct((B, N, OUT_F), jnp.float32),
        compiler_params=pltpu.CompilerParams(
            dimension_semantics=("parallel",)),
    )(adj_matrix, adj_matrix, xf, wbd_self, wbd_neigh, bbd_self, bbd_neigh)

    return out
